# baseline (speedup 1.0000x reference)
import functools

import jax
import jax.numpy as jnp
from jax import lax
from jax.experimental import pallas as pl
from jax.experimental.pallas import tpu as pltpu
from jax.experimental.pallas import tpu_sc as plsc

SPARSITY = 0.1
L = 16
NC = 2
NS = 16
NW = NC * NS
CHUNK = 8


def _row_select(rbuf, kbuf, hist, hsbuf, rb, n, k):
    nv = n // L
    lanes = lax.iota(jnp.int32, L)
    lane_off = lanes * 256
    ones_i = jnp.ones((L,), jnp.int32)
    zeros_i = jnp.zeros((L,), jnp.int32)
    scale = jnp.float32(n / k)

    def scan0(i, _):
        v = rbuf[pl.ds(rb + i * L, L)]
        kv = lax.bitcast_convert_type(v, jnp.int32) & jnp.int32(0x7FFFFFFF)
        kbuf[pl.ds(i * L, L)] = kv
        plsc.addupdate_scatter(hist, [(kv >> 23) + lane_off], ones_i)
        return 0

    def scan_l(i, args):
        shift, pshift, prefix = args
        kv = kbuf[pl.ds(i * L, L)]
        pm = (kv >> pshift) == prefix
        binv = (kv >> shift) & 255
        plsc.addupdate_scatter(hist, [binv + lane_off], ones_i, mask=pm)
        return args

    def reduce_clear():
        for v in range(16):
            def racc(lane, acc, v=v):
                sl = pl.ds(lane * 256 + v * L, L)
                h = hist[sl]
                hist[sl] = zeros_i
                return acc + h
            hsbuf[pl.ds(v * L, L)] = lax.fori_loop(0, 16, racc, zeros_i)

    def select(k_rem):
        def sbody(j, carry):
            running, nq = carry
            cv = hsbuf[pl.ds((15 - j) * L, L)]
            rc = plsc.cumsum(jnp.flip(cv, axis=0))
            qual = (rc + running) >= k_rem
            nq = nq + jnp.sum(qual.astype(jnp.int32))
            running = running + jnp.max(rc)
            return running, nq
        _, nq = lax.fori_loop(0, 16, sbody, (jnp.int32(0), jnp.int32(0)))
        b_star = nq - 1

        def cbody(v, acc):
            cv = hsbuf[pl.ds(v * L, L)]
            bins = lanes + v * L
            return acc + jnp.where(bins > b_star, cv, 0)
        count_above = jnp.sum(lax.fori_loop(0, 16, cbody, zeros_i))
        return b_star, count_above

    k_rem = jnp.int32(k)
    lax.fori_loop(0, nv, scan0, 0)
    reduce_clear()
    b0, ca = select(k_rem)
    k_rem = k_rem - ca
    p0 = b0

    lax.fori_loop(0, nv, scan_l, (jnp.int32(15), jnp.int32(23), p0))
    reduce_clear()
    b1, ca = select(k_rem)
    k_rem = k_rem - ca
    p1 = (p0 << 8) | b1

    lax.fori_loop(0, nv, scan_l, (jnp.int32(7), jnp.int32(15), p1))
    reduce_clear()
    b2, ca = select(k_rem)
    k_rem = k_rem - ca
    p2 = (p1 << 8) | b2

    lax.fori_loop(0, nv, scan_l, (jnp.int32(0), jnp.int32(7), p2))
    reduce_clear()
    b3, _ = select(k_rem)
    thr = (p2 << 7) | (b3 & 127)

    def outb(i, _):
        kv = kbuf[pl.ds(i * L, L)]
        v = rbuf[pl.ds(rb + i * L, L)]
        rbuf[pl.ds(rb + i * L, L)] = jnp.where(kv >= thr, v * scale,
                                               jnp.float32(0.0))
        return 0
    lax.fori_loop(0, nv, outb, 0)


def _make_sc_kernel(rows, n, k):
    rpw = rows // NW
    nchunk = rpw // CHUNK
    mesh = plsc.VectorSubcoreMesh(core_axis_name="c", subcore_axis_name="s",
                                  num_cores=NC, num_subcores=NS)

    @functools.partial(
        pl.kernel,
        out_type=jax.ShapeDtypeStruct((rows * n,), jnp.float32),
        mesh=mesh,
        compiler_params=pltpu.CompilerParams(needs_layout_passes=False),
        scratch_types=[
            pltpu.VMEM((CHUNK * n,), jnp.float32),
            pltpu.VMEM((n,), jnp.int32),
            pltpu.VMEM((16 * 256,), jnp.int32),
            pltpu.VMEM((256,), jnp.int32),
        ],
    )
    def sc_kernel(x_hbm, o_hbm, rbuf, kbuf, hist, hsbuf):
        cid = lax.axis_index("c")
        sid = lax.axis_index("s")
        wid = sid * NC + cid
        row0 = wid * rpw
        zeros_i = jnp.zeros((L,), jnp.int32)

        def z(i, _):
            hist[pl.ds(i * L, L)] = zeros_i
            return 0
        lax.fori_loop(0, 256, z, 0)

        def chunk(ch, _):
            base = (row0 + ch * CHUNK) * n
            pltpu.sync_copy(x_hbm.at[pl.ds(base, CHUNK * n)], rbuf)

            def rowloop(r, _):
                _row_select(rbuf, kbuf, hist, hsbuf, r * n, n, k)
                return 0
            lax.fori_loop(0, CHUNK, rowloop, 0)
            pltpu.sync_copy(rbuf, o_hbm.at[pl.ds(base, CHUNK * n)])
            return 0
        lax.fori_loop(0, nchunk, chunk, 0)

    return sc_kernel


def kernel(x):
    b, s, n = x.shape
    k = max(1, int(n * SPARSITY))
    rows = b * s
    out = _make_sc_kernel(rows, n, k)(x.reshape(rows * n))
    return out.reshape(b, s, n)


# speedup vs baseline: 1.2277x; 1.2277x over previous
import functools

import jax
import jax.numpy as jnp
from jax import lax
from jax.experimental import pallas as pl
from jax.experimental.pallas import tpu as pltpu
from jax.experimental.pallas import tpu_sc as plsc

SPARSITY = 0.1
L = 16
NC = 2
NS = 16
NW = NC * NS
CHUNK = 8
U = 8


def _row_select(rbuf, hist, hsbuf, hist16, cbuf, rb, n, k):
    nv = n // L
    lanes = lax.iota(jnp.int32, L)
    bank256 = lanes * 256
    bank16 = lanes * L
    ones_i = jnp.ones((L,), jnp.int32)
    zeros_i = jnp.zeros((L,), jnp.int32)
    sentinel = jnp.full((L,), 0x7FFFFFFF, jnp.int32)
    scale = jnp.float32(n / k)
    kmask = jnp.int32(0x7FFFFFFF)

    def keys_at(off):
        v = rbuf[pl.ds(off, L)]
        return lax.bitcast_convert_type(v, jnp.int32) & kmask, v

    def scan0(i, c):
        for u in range(U):
            kv, _ = keys_at(rb + (i * U + u) * L)
            plsc.addupdate_scatter(hist, [(kv >> 23) + bank256], ones_i)
        return c
    lax.fori_loop(0, nv // U, scan0, 0)

    def red0(v, c):
        acc = zeros_i
        for lane in range(16):
            sl = pl.ds(lane * 256 + v * L, L)
            acc = acc + hist[sl]
            hist[sl] = zeros_i
        hsbuf[pl.ds(v * L, L)] = acc
        return c
    lax.fori_loop(0, 16, red0, 0)

    k_rem = jnp.int32(k)

    def sel0_body(j, carry):
        running, nq, ca = carry
        cv = hsbuf[pl.ds((15 - j) * L, L)]
        rc = plsc.cumsum(jnp.flip(cv, axis=0))
        rcq = rc + running
        qual = rcq >= k_rem
        nq = nq + jnp.sum(qual.astype(jnp.int32))
        ca = jnp.maximum(ca, jnp.max(jnp.where(qual, 0, rcq)))
        running = running + jnp.max(rc)
        return running, nq, ca
    _, nq0, ca0 = lax.fori_loop(
        0, 16, sel0_body, (jnp.int32(0), jnp.int32(0), jnp.int32(0)))
    p = nq0 - 1
    k_rem = k_rem - ca0

    def scan1(i, off):
        for u in range(U):
            kv, _ = keys_at(rb + (i * U + u) * L)
            pm = (kv >> 23) == p
            plsc.addupdate_scatter(hist16, [((kv >> 19) & 15) + bank16],
                                   ones_i, mask=pm)
            plsc.store_compressed(cbuf.at[pl.ds(off, L)], kv, mask=pm)
            off = off + jnp.sum(pm.astype(jnp.int32))
        return off
    m = lax.fori_loop(0, nv // U, scan1, jnp.int32(0))
    cbuf[pl.ds(m, L)] = sentinel
    cbuf[pl.ds(m + L, L)] = sentinel

    def reduce16():
        acc = zeros_i
        for lane in range(16):
            sl = pl.ds(lane * L, L)
            acc = acc + hist16[sl]
            hist16[sl] = zeros_i
        return acc

    def sel16(acc, kr):
        rc = plsc.cumsum(jnp.flip(acc, axis=0))
        qual = rc >= kr
        nq = jnp.sum(qual.astype(jnp.int32))
        ca = jnp.max(jnp.where(qual, 0, rc))
        return nq - 1, ca

    b1, ca1 = sel16(reduce16(), k_rem)
    p = (p << 4) | b1
    k_rem = k_rem - ca1

    nv2 = (m + 2 * L - 1) // (2 * L)
    for shift in (15, 11, 7, 3):
        def scanc(i, c, shift=shift, p=p):
            for u in range(2):
                kv = cbuf[pl.ds((i * 2 + u) * L, L)]
                pm = (kv >> (shift + 4)) == p
                plsc.addupdate_scatter(hist16, [((kv >> shift) & 15) + bank16],
                                       ones_i, mask=pm)
            return c
        lax.fori_loop(0, nv2, scanc, 0)
        b, ca = sel16(reduce16(), k_rem)
        p = (p << 4) | b
        k_rem = k_rem - ca

    def scanf(i, c, p=p):
        for u in range(2):
            kv = cbuf[pl.ds((i * 2 + u) * L, L)]
            pm = (kv >> 3) == p
            plsc.addupdate_scatter(hist16, [(kv & 15) + bank16],
                                   ones_i, mask=pm)
        return c
    lax.fori_loop(0, nv2, scanf, 0)
    b6, _ = sel16(reduce16(), k_rem)
    thr = (p << 3) | (b6 & 7)

    def outb(i, c):
        for u in range(U):
            off = rb + (i * U + u) * L
            kv, v = keys_at(off)
            rbuf[pl.ds(off, L)] = jnp.where(kv >= thr, v * scale,
                                            jnp.float32(0.0))
        return c
    lax.fori_loop(0, nv // U, outb, 0)


def _make_sc_kernel(rows, n, k):
    rpw = rows // NW
    nchunk = rpw // CHUNK
    mesh = plsc.VectorSubcoreMesh(core_axis_name="c", subcore_axis_name="s",
                                  num_cores=NC, num_subcores=NS)

    @functools.partial(
        pl.kernel,
        out_type=jax.ShapeDtypeStruct((rows * n,), jnp.float32),
        mesh=mesh,
        compiler_params=pltpu.CompilerParams(needs_layout_passes=False),
        scratch_types=[
            pltpu.VMEM((CHUNK * n,), jnp.float32),
            pltpu.VMEM((16 * 256,), jnp.int32),
            pltpu.VMEM((256,), jnp.int32),
            pltpu.VMEM((256,), jnp.int32),
            pltpu.VMEM((n + 2 * L,), jnp.int32),
        ],
    )
    def sc_kernel(x_hbm, o_hbm, rbuf, hist, hsbuf, hist16, cbuf):
        cid = lax.axis_index("c")
        sid = lax.axis_index("s")
        wid = sid * NC + cid
        row0 = wid * rpw
        zeros_i = jnp.zeros((L,), jnp.int32)

        def z(i, _):
            hist[pl.ds(i * L, L)] = zeros_i
            return 0
        lax.fori_loop(0, 256, z, 0)
        def z16(i, _):
            hist16[pl.ds(i * L, L)] = zeros_i
            return 0
        lax.fori_loop(0, 16, z16, 0)

        def chunk(ch, _):
            base = (row0 + ch * CHUNK) * n
            pltpu.sync_copy(x_hbm.at[pl.ds(base, CHUNK * n)], rbuf)

            def rowloop(r, _):
                _row_select(rbuf, hist, hsbuf, hist16, cbuf, r * n, n, k)
                return 0
            lax.fori_loop(0, CHUNK, rowloop, 0)
            pltpu.sync_copy(rbuf, o_hbm.at[pl.ds(base, CHUNK * n)])
            return 0
        lax.fori_loop(0, nchunk, chunk, 0)

    return sc_kernel


def kernel(x):
    b, s, n = x.shape
    k = max(1, int(n * SPARSITY))
    rows = b * s
    out = _make_sc_kernel(rows, n, k)(x.reshape(rows * n))
    return out.reshape(b, s, n)
